# baseline (speedup 1.0000x reference)
import jax
import jax.numpy as jnp
from jax import lax
from jax.experimental import pallas as pl
from jax.experimental.pallas import tpu as pltpu
from jax.experimental.pallas import tpu_sc as plsc

PAGE = 64
BPB = 64
NBUF = 8
DEPTH = 4


def _cache_body(scal_ref, seq_sm, pi_sm, cache_hbm, cache_out,
                bufs, srcpg_buf, in_sems, out_sems, srcpg_sem):
    parent = scal_ref[0]
    fresh = scal_ref[2]
    src_len = seq_sm[parent]
    last_idx = jnp.maximum((src_len + PAGE - 1) // PAGE - 1, 0)
    has_partial = jnp.logical_and(src_len % PAGE != 0, src_len > 0)
    src_page = pi_sm[parent, last_idx]

    n_pages = cache_hbm.shape[0]
    nblk = n_pages // BPB

    srcpg_in = pltpu.make_async_copy(cache_hbm.at[pl.ds(src_page, 1)],
                                     srcpg_buf, srcpg_sem)
    srcpg_in.start()

    ins = [None] * nblk
    outs = [None] * nblk

    def start_in(i):
        b = i % NBUF
        c = pltpu.make_async_copy(cache_hbm.at[pl.ds(i * BPB, BPB)],
                                  bufs.at[b], in_sems.at[b])
        c.start()
        ins[i] = c

    def drain(j):
        b = j % NBUF
        ins[j].wait()
        blk_has_fresh = jnp.logical_and(
            has_partial,
            jnp.logical_and(fresh >= j * BPB, fresh < (j + 1) * BPB))

        @pl.when(blk_has_fresh)
        def _():
            bufs[b, pl.ds(fresh - j * BPB, 1)] = srcpg_buf[...]

        c = pltpu.make_async_copy(bufs.at[b], cache_out.at[pl.ds(j * BPB, BPB)],
                                  out_sems.at[b])
        c.start()
        outs[j] = c

    srcpg_in.wait()

    for i in range(nblk):
        if i >= NBUF:
            outs[i - NBUF].wait()
        start_in(i)
        j = i - DEPTH
        if j >= 0:
            drain(j)
    for j in range(max(nblk - DEPTH, 0), nblk):
        drain(j)
    for j in range(max(nblk - NBUF, 0), nblk):
        outs[j].wait()


def _cache_clone(scal, seq_lens, page_indices, cache):
    return pl.pallas_call(
        _cache_body,
        out_shape=jax.ShapeDtypeStruct(cache.shape, cache.dtype),
        in_specs=[
            pl.BlockSpec(memory_space=pltpu.SMEM),
            pl.BlockSpec(memory_space=pltpu.SMEM),
            pl.BlockSpec(memory_space=pltpu.SMEM),
            pl.BlockSpec(memory_space=pl.ANY),
        ],
        out_specs=pl.BlockSpec(memory_space=pl.ANY),
        scratch_shapes=[
            pltpu.VMEM((NBUF, BPB) + cache.shape[1:], cache.dtype),
            pltpu.VMEM((1,) + cache.shape[1:], cache.dtype),
            pltpu.SemaphoreType.DMA((NBUF,)),
            pltpu.SemaphoreType.DMA((NBUF,)),
            pltpu.SemaphoreType.DMA,
        ],
    )(scal, seq_lens, page_indices, cache)


def _state_clone(scal, tokens, seq_lens, page_indices):
    n_slots = tokens.shape[0]
    mesh = plsc.ScalarSubcoreMesh(axis_name="c", num_cores=2)
    rows_per_core = n_slots // 2

    @pl.kernel(
        out_type=(
            jax.ShapeDtypeStruct(tokens.shape, tokens.dtype),
            jax.ShapeDtypeStruct(seq_lens.shape, seq_lens.dtype),
            jax.ShapeDtypeStruct(page_indices.shape, page_indices.dtype),
        ),
        mesh=mesh,
        scratch_types=[
            pltpu.SMEM((3,), jnp.int32),
            pltpu.SMEM(seq_lens.shape, seq_lens.dtype),
            pltpu.SMEM((1, page_indices.shape[1]), page_indices.dtype),
            pltpu.SemaphoreType.DMA,
            pltpu.SemaphoreType.DMA,
            pltpu.SemaphoreType.DMA,
        ],
    )
    def state_kernel(scal_hbm, tokens_hbm, seq_hbm, pi_hbm, tok_out, seq_out,
                     pi_out, scal_sm, seq_sm, pirow_sm, sem, tok_sem, pi_sem):
        core = lax.axis_index("c")
        pltpu.async_copy(scal_hbm, scal_sm, sem).wait()
        pltpu.async_copy(seq_hbm, seq_sm, sem).wait()
        parent = scal_sm[0]
        child = scal_sm[1]
        fresh = scal_sm[2]
        src_len = seq_sm[parent]
        last_idx = jnp.maximum((src_len + PAGE - 1) // PAGE - 1, 0)
        has_partial = jnp.logical_and(src_len % PAGE != 0, src_len > 0)

        base = core * rows_per_core

        @pl.loop(0, rows_per_core)
        def _(r):
            i = base + r
            src = jnp.where(i == child, parent, i)
            pltpu.make_async_copy(tokens_hbm.at[pl.ds(src, 1)],
                                  tok_out.at[pl.ds(i, 1)], tok_sem).start()
            pltpu.make_async_copy(pi_hbm.at[pl.ds(src, 1)],
                                  pi_out.at[pl.ds(i, 1)], pi_sem).start()

        @pl.when(core == 0)
        def _():
            seq_sm[child] = src_len
            pltpu.async_copy(seq_sm, seq_out, sem).wait()

        @pl.loop(0, rows_per_core)
        def _(r):
            pltpu.make_async_copy(tokens_hbm.at[pl.ds(0, 1)],
                                  tok_out.at[pl.ds(0, 1)], tok_sem).wait()
            pltpu.make_async_copy(pi_hbm.at[pl.ds(0, 1)],
                                  pi_out.at[pl.ds(0, 1)], pi_sem).wait()

        owner = jnp.where(child >= rows_per_core, 1, 0)

        @pl.when(core == owner)
        def _():
            pltpu.async_copy(pi_hbm.at[pl.ds(parent, 1)], pirow_sm, sem).wait()

            @pl.when(has_partial)
            def _():
                pirow_sm[0, last_idx] = fresh

            pltpu.async_copy(pirow_sm, pi_out.at[pl.ds(child, 1)], sem).wait()

    return state_kernel(scal, tokens, seq_lens, page_indices)


def kernel(tokens, seq_lens, page_indices, cache, parent_local_id,
           child_local_id, fresh_page):
    scal = jnp.stack([
        jnp.asarray(parent_local_id, jnp.int32),
        jnp.asarray(child_local_id, jnp.int32),
        jnp.asarray(fresh_page, jnp.int32),
    ])
    cache_out = _cache_clone(scal, seq_lens, page_indices, cache)
    tokens_out, seq_out, pi_out = _state_clone(scal, tokens, seq_lens,
                                               page_indices)
    return tokens_out, seq_out, pi_out, cache_out


# speedup vs baseline: 1.4324x; 1.4324x over previous
import jax
import jax.numpy as jnp
from jax import lax
from jax.experimental import pallas as pl
from jax.experimental.pallas import tpu as pltpu
from jax.experimental.pallas import tpu_sc as plsc

PAGE = 64
BPB = 64
NBUF = 8
DEPTH = 4


def _cache_body(scal_ref, seq_sm, pi_sm, cache_hbm, cache_out,
                bufs, srcpg_buf, in_sems, out_sems, srcpg_sem):
    parent = scal_ref[0]
    fresh = scal_ref[2]
    src_len = seq_sm[parent]
    last_idx = jnp.maximum((src_len + PAGE - 1) // PAGE - 1, 0)
    has_partial = jnp.logical_and(src_len % PAGE != 0, src_len > 0)
    src_page = pi_sm[parent, last_idx]

    n_pages = cache_hbm.shape[0]
    nblk = n_pages // BPB

    srcpg_in = pltpu.make_async_copy(cache_hbm.at[pl.ds(src_page, 1)],
                                     srcpg_buf, srcpg_sem)
    srcpg_in.start()

    ins = [None] * nblk
    outs = [None] * nblk

    def start_in(i):
        b = i % NBUF
        c = pltpu.make_async_copy(cache_hbm.at[pl.ds(i * BPB, BPB)],
                                  bufs.at[b], in_sems.at[b])
        c.start()
        ins[i] = c

    def drain(j):
        b = j % NBUF
        ins[j].wait()
        blk_has_fresh = jnp.logical_and(
            has_partial,
            jnp.logical_and(fresh >= j * BPB, fresh < (j + 1) * BPB))

        @pl.when(blk_has_fresh)
        def _():
            bufs[b, pl.ds(fresh - j * BPB, 1)] = srcpg_buf[...]

        c = pltpu.make_async_copy(bufs.at[b], cache_out.at[pl.ds(j * BPB, BPB)],
                                  out_sems.at[b])
        c.start()
        outs[j] = c

    srcpg_in.wait()

    for i in range(nblk):
        if i >= NBUF:
            outs[i - NBUF].wait()
        start_in(i)
        j = i - DEPTH
        if j >= 0:
            drain(j)
    for j in range(max(nblk - DEPTH, 0), nblk):
        drain(j)
    for j in range(max(nblk - NBUF, 0), nblk):
        outs[j].wait()


def _cache_clone(scal, seq_lens, page_indices, cache):
    return pl.pallas_call(
        _cache_body,
        out_shape=jax.ShapeDtypeStruct(cache.shape, cache.dtype),
        in_specs=[
            pl.BlockSpec(memory_space=pltpu.SMEM),
            pl.BlockSpec(memory_space=pltpu.SMEM),
            pl.BlockSpec(memory_space=pltpu.SMEM),
            pl.BlockSpec(memory_space=pl.ANY),
        ],
        out_specs=pl.BlockSpec(memory_space=pl.ANY),
        scratch_shapes=[
            pltpu.VMEM((NBUF, BPB) + cache.shape[1:], cache.dtype),
            pltpu.VMEM((1,) + cache.shape[1:], cache.dtype),
            pltpu.SemaphoreType.DMA((NBUF,)),
            pltpu.SemaphoreType.DMA((NBUF,)),
            pltpu.SemaphoreType.DMA,
        ],
    )(scal, seq_lens, page_indices, cache)


def _state_clone(scal, tokens, seq_lens, page_indices):
    n_slots = tokens.shape[0]
    mesh = plsc.ScalarSubcoreMesh(axis_name="c", num_cores=2)
    rows_per_core = n_slots // 2

    @pl.kernel(
        out_type=(
            jax.ShapeDtypeStruct(tokens.shape, tokens.dtype),
            jax.ShapeDtypeStruct(seq_lens.shape, seq_lens.dtype),
            jax.ShapeDtypeStruct(page_indices.shape, page_indices.dtype),
        ),
        mesh=mesh,
        scratch_types=[
            pltpu.SMEM((3,), jnp.int32),
            pltpu.SMEM(seq_lens.shape, seq_lens.dtype),
            pltpu.SMEM((1, page_indices.shape[1]), page_indices.dtype),
            pltpu.VMEM_SHARED((rows_per_core, tokens.shape[1]), tokens.dtype),
            pltpu.VMEM_SHARED((rows_per_core, page_indices.shape[1]),
                              page_indices.dtype),
            pltpu.SemaphoreType.DMA,
            pltpu.SemaphoreType.DMA,
            pltpu.SemaphoreType.DMA,
        ],
    )
    def state_kernel(scal_hbm, tokens_hbm, seq_hbm, pi_hbm, tok_out, seq_out,
                     pi_out, scal_sm, seq_sm, pirow_sm, tok_sp, pi_sp, sem,
                     tok_sem, pi_sem):
        core = lax.axis_index("c")
        base = core * rows_per_core

        tok_in = pltpu.make_async_copy(
            tokens_hbm.at[pl.ds(base, rows_per_core)], tok_sp, tok_sem)
        tok_in.start()
        pi_in = pltpu.make_async_copy(
            pi_hbm.at[pl.ds(base, rows_per_core)], pi_sp, pi_sem)
        pi_in.start()

        pltpu.async_copy(scal_hbm, scal_sm, sem).wait()
        pltpu.async_copy(seq_hbm, seq_sm, sem).wait()
        parent = scal_sm[0]
        child = scal_sm[1]
        fresh = scal_sm[2]
        src_len = seq_sm[parent]
        last_idx = jnp.maximum((src_len + PAGE - 1) // PAGE - 1, 0)
        has_partial = jnp.logical_and(src_len % PAGE != 0, src_len > 0)

        @pl.when(core == 0)
        def _():
            seq_sm[child] = src_len
            pltpu.async_copy(seq_sm, seq_out, sem).wait()

        tok_in.wait()
        pi_in.wait()

        owner = jnp.where(child >= rows_per_core, 1, 0)

        @pl.when(core == owner)
        def _():
            pltpu.async_copy(tokens_hbm.at[pl.ds(parent, 1)],
                             tok_sp.at[pl.ds(child - base, 1)], sem).wait()

        tok_w = pltpu.make_async_copy(
            tok_sp, tok_out.at[pl.ds(base, rows_per_core)], tok_sem)
        tok_w.start()
        pi_w = pltpu.make_async_copy(
            pi_sp, pi_out.at[pl.ds(base, rows_per_core)], pi_sem)
        pi_w.start()
        tok_w.wait()
        pi_w.wait()

        @pl.when(core == owner)
        def _():
            pltpu.async_copy(pi_hbm.at[pl.ds(parent, 1)], pirow_sm, sem).wait()

            @pl.when(has_partial)
            def _():
                pirow_sm[0, last_idx] = fresh

            pltpu.async_copy(pirow_sm, pi_out.at[pl.ds(child, 1)], sem).wait()

    return state_kernel(scal, tokens, seq_lens, page_indices)


def kernel(tokens, seq_lens, page_indices, cache, parent_local_id,
           child_local_id, fresh_page):
    scal = jnp.stack([
        jnp.asarray(parent_local_id, jnp.int32),
        jnp.asarray(child_local_id, jnp.int32),
        jnp.asarray(fresh_page, jnp.int32),
    ])
    cache_out = _cache_clone(scal, seq_lens, page_indices, cache)
    tokens_out, seq_out, pi_out = _state_clone(scal, tokens, seq_lens,
                                               page_indices)
    return tokens_out, seq_out, pi_out, cache_out
